# baseline (speedup 1.0000x reference)
import functools

import jax
import jax.numpy as jnp
from jax import lax
from jax.experimental import pallas as pl
from jax.experimental.pallas import tpu as pltpu
from jax.experimental.pallas import tpu_sc as plsc

N = 100000
H = 128
G = 1024
B = 4000
NB = N // B
GC = 128
NGC = G // GC

NC = 2
NS = 16
NW = NC * NS
CH = 80
NCH = N // CH
MAXJ = (NCH + NW - 1) // NW

_dot = functools.partial(jnp.dot, preferred_element_type=jnp.float32)


def _hilo(a):
    hi = a.astype(jnp.bfloat16)
    lo = (a - hi.astype(jnp.float32)).astype(jnp.bfloat16)
    return hi, lo


def _mm3(a, b):
    ah, al = _hilo(a)
    bh, bl = _hilo(b)
    return _dot(ah, bh) + (_dot(al, bh) + _dot(ah, bl))


def _precompute_body(x_ref, brow_ref, w1_ref, b1_ref,
                     xw1_ref, sums_ref, cnt_ref):
    i = pl.program_id(0)

    @pl.when(i == 0)
    def _():
        sums_ref[...] = jnp.zeros_like(sums_ref)
        cnt_ref[...] = jnp.zeros_like(cnt_ref)

    xb = x_ref[...]
    xw1_ref[...] = _mm3(xb, w1_ref[...]) + b1_ref[...]

    brow = brow_ref[0]
    bmin = brow_ref[0, 0, 0]
    bmax = brow_ref[0, 0, B - 1]
    xh, xl = _hilo(xb)
    for c in range(NGC):
        base = c * GC

        @pl.when((bmax >= base) & (bmin < base + GC))
        def _(base=base):
            iog = lax.broadcasted_iota(jnp.int32, (GC, B), 0) + base
            ohg = (iog == brow)
            ohb = ohg.astype(jnp.bfloat16)
            sums_ref[base:base + GC, :] += _dot(ohb, xh) + _dot(ohb, xl)
            cnt_ref[base:base + GC, :] += jnp.sum(
                ohg.astype(jnp.float32), axis=1, keepdims=True)


def _meanw1_body(sums_ref, cnt_ref, w1_ref, repr_ref, rw1_ref):
    mean = sums_ref[...] / jnp.maximum(cnt_ref[...], 1.0)
    repr_ref[...] = mean
    rw1_ref[...] = _mm3(mean, w1_ref[...])


def _gru_body(wsums_ref, cnt_ref, prev_ref, wihT_ref, whhT_ref,
              bih_ref, bhh_ref, w1_ref, repr_ref, rw1_ref):
    wsums = wsums_ref[0] + wsums_ref[1]
    mean = wsums / jnp.maximum(cnt_ref[...], 1.0)
    prev = prev_ref[...]
    gi = _mm3(mean, wihT_ref[...]) + bih_ref[...]
    gh = _mm3(prev, whhT_ref[...]) + bhh_ref[...]
    r = jax.nn.sigmoid(gi[:, :H] + gh[:, :H])
    z = jax.nn.sigmoid(gi[:, H:2 * H] + gh[:, H:2 * H])
    n = jnp.tanh(gi[:, 2 * H:] + r * gh[:, 2 * H:])
    new = jnp.maximum((1.0 - z) * n + z * prev, 0.0)
    repr_ref[...] = new
    rw1_ref[...] = _mm3(new, w1_ref[...])


def _sc_gatepass_body(xw1_hbm, x_hbm, batch_hbm, rw1_hbm, w2_hbm, b2_hbm,
                      zeros_hbm, out_hbm,
                      idx_v, xw_v, x_v, r_v, w_v, w2_v, b2_v, accum, sem):
    c = lax.axis_index("c")
    s = lax.axis_index("s")
    w = s * NC + c

    pltpu.sync_copy(zeros_hbm.at[pl.ds(s * (G // NS), G // NS)],
                    accum.at[pl.ds(s * (G // NS), G // NS)])
    pltpu.sync_copy(w2_hbm, w2_v)
    pltpu.sync_copy(b2_hbm, b2_v)
    plsc.subcore_barrier()
    b2vec = b2_v[...]
    lanes = lax.broadcasted_iota(jnp.int32, (16,), 0)
    rots = [(lanes + sh) % 16 for sh in (8, 4, 2, 1)]

    def chunk_body(j, carry):
        cid = j * NW + w

        @pl.when(cid < NCH)
        def _():
            base = cid * CH
            pltpu.sync_copy(batch_hbm.at[pl.ds(base, CH)], idx_v)
            pltpu.sync_copy(xw1_hbm.at[pl.ds(base, CH)], xw_v)
            pltpu.sync_copy(x_hbm.at[pl.ds(base, CH)], x_v)
            pltpu.async_copy(rw1_hbm.at[idx_v], r_v, sem).wait()

            def row_body(i, carry2):
                acc = jnp.zeros((16,), jnp.float32)
                for l in range(8):
                    a = xw_v[i, pl.ds(l * 16, 16)] + r_v[i, pl.ds(l * 16, 16)]
                    acc = acc + jnp.maximum(a, 0.0) * w2_v[pl.ds(l * 16, 16)]
                for rot in rots:
                    acc = acc + lax.gather(
                        acc, rot[:, None],
                        lax.GatherDimensionNumbers(
                            offset_dims=(), collapsed_slice_dims=(0,),
                            start_index_map=(0,)),
                        slice_sizes=(1,),
                        mode=lax.GatherScatterMode.PROMISE_IN_BOUNDS)
                z = acc + b2vec
                g = 1.0 / (1.0 + jnp.exp(-z))
                for l in range(8):
                    w_v[i, pl.ds(l * 16, 16)] = x_v[i, pl.ds(l * 16, 16)] * g
                return carry2

            lax.fori_loop(0, CH, row_body, 0, unroll=2)
            pltpu.sync_copy(w_v, accum.at[idx_v], add=True)

        return carry

    lax.fori_loop(0, MAXJ, chunk_body, 0)
    plsc.subcore_barrier()

    off = c * G + s * (G // NS)
    pltpu.sync_copy(accum.at[pl.ds(s * (G // NS), G // NS)],
                    out_hbm.at[pl.ds(off, G // NS)])


def kernel(x, batch, gate_W1, gate_b1, gate_W2, gate_b2,
           W_ih, W_hh, b_ih, b_hh):
    batch = batch.astype(jnp.int32)
    brow = batch.reshape(NB, 1, B)
    b1r = gate_b1.reshape(1, H)
    w2flat = gate_W2.reshape(H)
    b2v = jnp.broadcast_to(gate_b2.reshape(1), (16,))
    wihT = W_ih.T
    whhT = W_hh.T
    bihr = b_ih.reshape(1, 3 * H)
    bhhr = b_hh.reshape(1, 3 * H)
    zeros_gh = jnp.zeros((G, H), jnp.float32)

    f32 = jnp.float32
    const = lambda shape: pl.BlockSpec(shape, lambda i: tuple(0 for _ in shape))

    xw1, sums0, cnt = pl.pallas_call(
        _precompute_body,
        grid=(NB,),
        in_specs=[
            pl.BlockSpec((B, H), lambda i: (i, 0)),
            pl.BlockSpec((1, 1, B), lambda i: (i, 0, 0)),
            const((H, H)),
            const((1, H)),
        ],
        out_specs=[
            pl.BlockSpec((B, H), lambda i: (i, 0)),
            const((G, H)),
            const((G, 1)),
        ],
        out_shape=[
            jax.ShapeDtypeStruct((N, H), f32),
            jax.ShapeDtypeStruct((G, H), f32),
            jax.ShapeDtypeStruct((G, 1), f32),
        ],
    )(x, brow, gate_W1, b1r)

    repr_, rw1 = pl.pallas_call(
        _meanw1_body,
        grid=(1,),
        in_specs=[const((G, H)), const((G, 1)), const((H, H))],
        out_specs=[const((G, H)), const((G, H))],
        out_shape=[jax.ShapeDtypeStruct((G, H), f32),
                   jax.ShapeDtypeStruct((G, H), f32)],
    )(sums0, cnt, gate_W1)

    mesh = plsc.VectorSubcoreMesh(core_axis_name="c", subcore_axis_name="s")
    gatepass = pl.kernel(
        _sc_gatepass_body,
        out_type=jax.ShapeDtypeStruct((NC * G, H), f32),
        mesh=mesh,
        scratch_types=[
            pltpu.VMEM((CH,), jnp.int32),
            pltpu.VMEM((CH, H), f32),
            pltpu.VMEM((CH, H), f32),
            pltpu.VMEM((CH, H), f32),
            pltpu.VMEM((CH, H), f32),
            pltpu.VMEM((H,), f32),
            pltpu.VMEM((16,), f32),
            pltpu.VMEM_SHARED((G, H), f32),
            pltpu.SemaphoreType.DMA,
        ],
    )

    gru = pl.pallas_call(
        _gru_body,
        grid=(1,),
        in_specs=[const((NC, G, H)), const((G, 1)), const((G, H)),
                  const((H, 3 * H)), const((H, 3 * H)),
                  const((1, 3 * H)), const((1, 3 * H)), const((H, H))],
        out_specs=[const((G, H)), const((G, H))],
        out_shape=[jax.ShapeDtypeStruct((G, H), f32),
                   jax.ShapeDtypeStruct((G, H), f32)],
    )

    for _ in range(2):
        wsums = gatepass(xw1, x, batch, rw1, w2flat, b2v, zeros_gh)
        repr_, rw1 = gru(wsums.reshape(NC, G, H), cnt, repr_,
                         wihT, whhT, bihr, bhhr, gate_W1)

    return repr_


# speedup vs baseline: 1.2638x; 1.2638x over previous
import functools

import jax
import jax.numpy as jnp
from jax import lax
from jax.experimental import pallas as pl
from jax.experimental.pallas import tpu as pltpu
from jax.experimental.pallas import tpu_sc as plsc

N = 100000
H = 128
G = 1024
B = 4000
NB = N // B
GC = 128
NGC = G // GC

NC = 2
NS = 16
NW = NC * NS
CH = 80
NCH = N // CH
MAXJ = (NCH + NW - 1) // NW

_dot = functools.partial(jnp.dot, preferred_element_type=jnp.float32)


def _hilo(a):
    hi = a.astype(jnp.bfloat16)
    lo = (a - hi.astype(jnp.float32)).astype(jnp.bfloat16)
    return hi, lo


def _mm3(a, b):
    ah, al = _hilo(a)
    bh, bl = _hilo(b)
    return _dot(ah, bh) + (_dot(al, bh) + _dot(ah, bl))


def _precompute_body(x_ref, brow_ref, w1_ref, b1_ref,
                     xw1_ref, sums_ref, cnt_ref):
    i = pl.program_id(0)

    @pl.when(i == 0)
    def _():
        sums_ref[...] = jnp.zeros_like(sums_ref)
        cnt_ref[...] = jnp.zeros_like(cnt_ref)

    xb = x_ref[...]
    xw1_ref[...] = _mm3(xb, w1_ref[...]) + b1_ref[...]

    brow = brow_ref[0]
    bmin = brow_ref[0, 0, 0]
    bmax = brow_ref[0, 0, B - 1]
    xh, xl = _hilo(xb)
    for c in range(NGC):
        base = c * GC

        @pl.when((bmax >= base) & (bmin < base + GC))
        def _(base=base):
            iog = lax.broadcasted_iota(jnp.int32, (GC, B), 0) + base
            ohg = (iog == brow)
            ohb = ohg.astype(jnp.bfloat16)
            sums_ref[base:base + GC, :] += _dot(ohb, xh) + _dot(ohb, xl)
            cnt_ref[base:base + GC, :] += jnp.sum(
                ohg.astype(jnp.float32), axis=1, keepdims=True)


def _meanw1_body(sums_ref, cnt_ref, w1_ref, repr_ref, rw1_ref):
    mean = sums_ref[...] / jnp.maximum(cnt_ref[...], 1.0)
    repr_ref[...] = mean
    rw1_ref[...] = _mm3(mean, w1_ref[...])


def _gru_body(wsums_ref, cnt_ref, prev_ref, wihT_ref, whhT_ref,
              bih_ref, bhh_ref, w1_ref, repr_ref, rw1_ref):
    wsums = wsums_ref[0] + wsums_ref[1]
    mean = wsums / jnp.maximum(cnt_ref[...], 1.0)
    prev = prev_ref[...]
    gi = _mm3(mean, wihT_ref[...]) + bih_ref[...]
    gh = _mm3(prev, whhT_ref[...]) + bhh_ref[...]
    r = jax.nn.sigmoid(gi[:, :H] + gh[:, :H])
    z = jax.nn.sigmoid(gi[:, H:2 * H] + gh[:, H:2 * H])
    n = jnp.tanh(gi[:, 2 * H:] + r * gh[:, 2 * H:])
    new = jnp.maximum((1.0 - z) * n + z * prev, 0.0)
    repr_ref[...] = new
    rw1_ref[...] = _mm3(new, w1_ref[...])


def _sc_gatepass_body(xw1_hbm, x_hbm, batch_hbm, rw1_hbm, w2_hbm, b2_hbm,
                      zeros_hbm, out_hbm,
                      idx_a, xw_a, x_a, idx_b, xw_b, x_b,
                      r_v, w_v, w2_v, b2_v, accum, sem_a, sem_b, sem_g):
    c = lax.axis_index("c")
    s = lax.axis_index("s")
    w = s * NC + c

    pltpu.sync_copy(zeros_hbm.at[pl.ds(s * (G // NS), G // NS)],
                    accum.at[pl.ds(s * (G // NS), G // NS)])
    pltpu.sync_copy(w2_hbm, w2_v)
    pltpu.sync_copy(b2_hbm, b2_v)
    plsc.subcore_barrier()
    b2vec = b2_v[...]
    lanes = lax.broadcasted_iota(jnp.int32, (16,), 0)
    rots = [(lanes + sh) % 16 for sh in (8, 4, 2, 1)]

    def _in_copies(j, idxv, xwv, xv, sem):
        base = (j * NW + w) * CH
        return (pltpu.make_async_copy(batch_hbm.at[pl.ds(base, CH)], idxv, sem),
                pltpu.make_async_copy(xw1_hbm.at[pl.ds(base, CH)], xwv, sem),
                pltpu.make_async_copy(x_hbm.at[pl.ds(base, CH)], xv, sem))

    def _start_in(j, idxv, xwv, xv, sem):
        for cp in _in_copies(j, idxv, xwv, xv, sem):
            cp.start()

    def _wait_in(j, idxv, xwv, xv, sem):
        for cp in _in_copies(j, idxv, xwv, xv, sem):
            cp.wait()

    def _process(idxv, xwv, xv):
        pltpu.async_copy(rw1_hbm.at[idxv], r_v, sem_g).wait()

        def row_body(i, carry2):
            acc = jnp.zeros((16,), jnp.float32)
            for l in range(8):
                a = xwv[i, pl.ds(l * 16, 16)] + r_v[i, pl.ds(l * 16, 16)]
                acc = acc + jnp.maximum(a, 0.0) * w2_v[pl.ds(l * 16, 16)]
            for rot in rots:
                acc = acc + lax.gather(
                    acc, rot[:, None],
                    lax.GatherDimensionNumbers(
                        offset_dims=(), collapsed_slice_dims=(0,),
                        start_index_map=(0,)),
                    slice_sizes=(1,),
                    mode=lax.GatherScatterMode.PROMISE_IN_BOUNDS)
            z = acc + b2vec
            g = 1.0 / (1.0 + jnp.exp(-z))
            for l in range(8):
                w_v[i, pl.ds(l * 16, 16)] = xv[i, pl.ds(l * 16, 16)] * g
            return carry2

        lax.fori_loop(0, CH, row_body, 0, unroll=2)
        pltpu.sync_copy(w_v, accum.at[idxv], add=True)

    _start_in(0, idx_a, xw_a, x_a, sem_a)

    def pair_body(k, carry):
        j0 = 2 * k
        _wait_in(j0, idx_a, xw_a, x_a, sem_a)
        _start_in(j0 + 1, idx_b, xw_b, x_b, sem_b)
        _process(idx_a, xw_a, x_a)
        _wait_in(j0 + 1, idx_b, xw_b, x_b, sem_b)
        _start_in(j0 + 2, idx_a, xw_a, x_a, sem_a)
        _process(idx_b, xw_b, x_b)
        return carry

    lax.fori_loop(0, (MAXJ - 2) // 2, pair_body, 0)
    _wait_in(MAXJ - 2, idx_a, xw_a, x_a, sem_a)
    _process(idx_a, xw_a, x_a)

    @pl.when(39 * NW + w < NCH)
    def _():
        _start_in(MAXJ - 1, idx_b, xw_b, x_b, sem_b)
        _wait_in(MAXJ - 1, idx_b, xw_b, x_b, sem_b)
        _process(idx_b, xw_b, x_b)

    plsc.subcore_barrier()

    off = c * G + s * (G // NS)
    pltpu.sync_copy(accum.at[pl.ds(s * (G // NS), G // NS)],
                    out_hbm.at[pl.ds(off, G // NS)])


def kernel(x, batch, gate_W1, gate_b1, gate_W2, gate_b2,
           W_ih, W_hh, b_ih, b_hh):
    batch = batch.astype(jnp.int32)
    brow = batch.reshape(NB, 1, B)
    b1r = gate_b1.reshape(1, H)
    w2flat = gate_W2.reshape(H)
    b2v = jnp.broadcast_to(gate_b2.reshape(1), (16,))
    wihT = W_ih.T
    whhT = W_hh.T
    bihr = b_ih.reshape(1, 3 * H)
    bhhr = b_hh.reshape(1, 3 * H)
    zeros_gh = jnp.zeros((G, H), jnp.float32)

    f32 = jnp.float32
    const = lambda shape: pl.BlockSpec(shape, lambda i: tuple(0 for _ in shape))

    xw1, sums0, cnt = pl.pallas_call(
        _precompute_body,
        grid=(NB,),
        in_specs=[
            pl.BlockSpec((B, H), lambda i: (i, 0)),
            pl.BlockSpec((1, 1, B), lambda i: (i, 0, 0)),
            const((H, H)),
            const((1, H)),
        ],
        out_specs=[
            pl.BlockSpec((B, H), lambda i: (i, 0)),
            const((G, H)),
            const((G, 1)),
        ],
        out_shape=[
            jax.ShapeDtypeStruct((N, H), f32),
            jax.ShapeDtypeStruct((G, H), f32),
            jax.ShapeDtypeStruct((G, 1), f32),
        ],
    )(x, brow, gate_W1, b1r)

    repr_, rw1 = pl.pallas_call(
        _meanw1_body,
        grid=(1,),
        in_specs=[const((G, H)), const((G, 1)), const((H, H))],
        out_specs=[const((G, H)), const((G, H))],
        out_shape=[jax.ShapeDtypeStruct((G, H), f32),
                   jax.ShapeDtypeStruct((G, H), f32)],
    )(sums0, cnt, gate_W1)

    mesh = plsc.VectorSubcoreMesh(core_axis_name="c", subcore_axis_name="s")
    gatepass = pl.kernel(
        _sc_gatepass_body,
        out_type=jax.ShapeDtypeStruct((NC * G, H), f32),
        mesh=mesh,
        scratch_types=[
            pltpu.VMEM((CH,), jnp.int32),
            pltpu.VMEM((CH, H), f32),
            pltpu.VMEM((CH, H), f32),
            pltpu.VMEM((CH,), jnp.int32),
            pltpu.VMEM((CH, H), f32),
            pltpu.VMEM((CH, H), f32),
            pltpu.VMEM((CH, H), f32),
            pltpu.VMEM((CH, H), f32),
            pltpu.VMEM((H,), f32),
            pltpu.VMEM((16,), f32),
            pltpu.VMEM_SHARED((G, H), f32),
            pltpu.SemaphoreType.DMA,
            pltpu.SemaphoreType.DMA,
            pltpu.SemaphoreType.DMA,
        ],
    )

    gru = pl.pallas_call(
        _gru_body,
        grid=(1,),
        in_specs=[const((NC, G, H)), const((G, 1)), const((G, H)),
                  const((H, 3 * H)), const((H, 3 * H)),
                  const((1, 3 * H)), const((1, 3 * H)), const((H, H))],
        out_specs=[const((G, H)), const((G, H))],
        out_shape=[jax.ShapeDtypeStruct((G, H), f32),
                   jax.ShapeDtypeStruct((G, H), f32)],
    )

    for _ in range(2):
        wsums = gatepass(xw1, x, batch, rw1, w2flat, b2v, zeros_gh)
        repr_, rw1 = gru(wsums.reshape(NC, G, H), cnt, repr_,
                         wihT, whhT, bihr, bhhr, gate_W1)

    return repr_


# speedup vs baseline: 1.9896x; 1.5743x over previous
import functools

import jax
import jax.numpy as jnp
from jax import lax
from jax.experimental import pallas as pl
from jax.experimental.pallas import tpu as pltpu
from jax.experimental.pallas import tpu_sc as plsc

N = 100000
H = 128
G = 1024
B = 4000
NB = N // B
GC = 128
NGC = G // GC

NC = 2
NS = 16
NW = NC * NS
CH = 80
NCH = N // CH
MAXJ = (NCH + NW - 1) // NW

_dot = functools.partial(jnp.dot, preferred_element_type=jnp.float32)


def _hilo(a):
    hi = a.astype(jnp.bfloat16)
    lo = (a - hi.astype(jnp.float32)).astype(jnp.bfloat16)
    return hi, lo


def _mm3(a, b):
    ah, al = _hilo(a)
    bh, bl = _hilo(b)
    return _dot(ah, bh) + (_dot(al, bh) + _dot(ah, bl))


def _precompute_body(x_ref, brow_ref, w1_ref, b1_ref,
                     xw1_ref, sums_ref, cnt_ref):
    i = pl.program_id(0)

    @pl.when(i == 0)
    def _():
        sums_ref[...] = jnp.zeros_like(sums_ref)
        cnt_ref[...] = jnp.zeros_like(cnt_ref)

    xb = x_ref[...]
    xw1_ref[...] = _mm3(xb, w1_ref[...]) + b1_ref[...]

    brow = brow_ref[0]
    bmin = brow_ref[0, 0, 0]
    bmax = brow_ref[0, 0, B - 1]
    xh, xl = _hilo(xb)
    for c in range(NGC):
        base = c * GC

        @pl.when((bmax >= base) & (bmin < base + GC))
        def _(base=base):
            iog = lax.broadcasted_iota(jnp.int32, (GC, B), 0) + base
            ohg = (iog == brow)
            ohb = ohg.astype(jnp.bfloat16)
            sums_ref[base:base + GC, :] += _dot(ohb, xh) + _dot(ohb, xl)
            cnt_ref[base:base + GC, :] += jnp.sum(
                ohg.astype(jnp.float32), axis=1, keepdims=True)


def _meanw1_body(sums_ref, cnt_ref, w1_ref, repr_ref, rw1_ref):
    mean = sums_ref[...] / jnp.maximum(cnt_ref[...], 1.0)
    repr_ref[...] = mean
    rw1_ref[...] = _mm3(mean, w1_ref[...])


def _gru_body(wsums_ref, cnt_ref, prev_ref, wihT_ref, whhT_ref,
              bih_ref, bhh_ref, w1_ref, repr_ref, rw1_ref):
    wsums = wsums_ref[0] + wsums_ref[1]
    mean = wsums / jnp.maximum(cnt_ref[...], 1.0)
    prev = prev_ref[...]
    gi = _mm3(mean, wihT_ref[...]) + bih_ref[...]
    gh = _mm3(prev, whhT_ref[...]) + bhh_ref[...]
    r = jax.nn.sigmoid(gi[:, :H] + gh[:, :H])
    z = jax.nn.sigmoid(gi[:, H:2 * H] + gh[:, H:2 * H])
    n = jnp.tanh(gi[:, 2 * H:] + r * gh[:, 2 * H:])
    new = jnp.maximum((1.0 - z) * n + z * prev, 0.0)
    repr_ref[...] = new
    rw1_ref[...] = _mm3(new, w1_ref[...])


def _sc_gatepass_body(xw1_hbm, x_hbm, batch_hbm, rw1_hbm, w2_hbm, b2_hbm,
                      zeros_hbm, out_hbm,
                      idx_a, xw_a, x_a, idx_b, xw_b, x_b,
                      r_v, w_v, w2_v, b2_v, accum, sem_a, sem_b, sem_g):
    c = lax.axis_index("c")
    s = lax.axis_index("s")
    w = s * NC + c

    pltpu.sync_copy(zeros_hbm.at[pl.ds(s * (G // NS), G // NS)],
                    accum.at[pl.ds(s * (G // NS), G // NS)])
    pltpu.sync_copy(w2_hbm, w2_v)
    pltpu.sync_copy(b2_hbm, b2_v)
    plsc.subcore_barrier()
    b2vec = b2_v[...]
    lanes = lax.broadcasted_iota(jnp.int32, (16,), 0)
    rots = [(lanes + sh) % 16 for sh in (8, 4, 2, 1)]

    def _in_copies(j, idxv, xwv, xv, sem):
        base = (j * NW + w) * CH
        return (pltpu.make_async_copy(batch_hbm.at[pl.ds(base, CH)], idxv, sem),
                pltpu.make_async_copy(xw1_hbm.at[pl.ds(base, CH)], xwv, sem),
                pltpu.make_async_copy(x_hbm.at[pl.ds(base, CH)], xv, sem))

    def _start_in(j, idxv, xwv, xv, sem):
        for cp in _in_copies(j, idxv, xwv, xv, sem):
            cp.start()

    def _wait_in(j, idxv, xwv, xv, sem):
        for cp in _in_copies(j, idxv, xwv, xv, sem):
            cp.wait()

    def _process(idxv, xwv, xv):
        pltpu.async_copy(rw1_hbm.at[idxv], r_v, sem_g).wait()

        @plsc.parallel_loop(0, CH, 1, unroll=4)
        def _row(i):
            acc = jnp.zeros((16,), jnp.float32)
            for l in range(8):
                a = xwv[i, pl.ds(l * 16, 16)] + r_v[i, pl.ds(l * 16, 16)]
                acc = acc + jnp.maximum(a, 0.0) * w2_v[pl.ds(l * 16, 16)]
            for rot in rots:
                acc = acc + lax.gather(
                    acc, rot[:, None],
                    lax.GatherDimensionNumbers(
                        offset_dims=(), collapsed_slice_dims=(0,),
                        start_index_map=(0,)),
                    slice_sizes=(1,),
                    mode=lax.GatherScatterMode.PROMISE_IN_BOUNDS)
            z = acc + b2vec
            g = 1.0 / (1.0 + jnp.exp(-z))
            for l in range(8):
                w_v[i, pl.ds(l * 16, 16)] = xv[i, pl.ds(l * 16, 16)] * g
        pltpu.sync_copy(w_v, accum.at[idxv], add=True)

    _start_in(0, idx_a, xw_a, x_a, sem_a)

    def pair_body(k, carry):
        j0 = 2 * k
        _wait_in(j0, idx_a, xw_a, x_a, sem_a)
        _start_in(j0 + 1, idx_b, xw_b, x_b, sem_b)
        _process(idx_a, xw_a, x_a)
        _wait_in(j0 + 1, idx_b, xw_b, x_b, sem_b)
        _start_in(j0 + 2, idx_a, xw_a, x_a, sem_a)
        _process(idx_b, xw_b, x_b)
        return carry

    lax.fori_loop(0, (MAXJ - 2) // 2, pair_body, 0)
    _wait_in(MAXJ - 2, idx_a, xw_a, x_a, sem_a)
    _process(idx_a, xw_a, x_a)

    @pl.when(39 * NW + w < NCH)
    def _():
        _start_in(MAXJ - 1, idx_b, xw_b, x_b, sem_b)
        _wait_in(MAXJ - 1, idx_b, xw_b, x_b, sem_b)
        _process(idx_b, xw_b, x_b)

    plsc.subcore_barrier()

    off = c * G + s * (G // NS)
    pltpu.sync_copy(accum.at[pl.ds(s * (G // NS), G // NS)],
                    out_hbm.at[pl.ds(off, G // NS)])


def kernel(x, batch, gate_W1, gate_b1, gate_W2, gate_b2,
           W_ih, W_hh, b_ih, b_hh):
    batch = batch.astype(jnp.int32)
    brow = batch.reshape(NB, 1, B)
    b1r = gate_b1.reshape(1, H)
    w2flat = gate_W2.reshape(H)
    b2v = jnp.broadcast_to(gate_b2.reshape(1), (16,))
    wihT = W_ih.T
    whhT = W_hh.T
    bihr = b_ih.reshape(1, 3 * H)
    bhhr = b_hh.reshape(1, 3 * H)
    zeros_gh = jnp.zeros((G, H), jnp.float32)

    f32 = jnp.float32
    const = lambda shape: pl.BlockSpec(shape, lambda i: tuple(0 for _ in shape))

    xw1, sums0, cnt = pl.pallas_call(
        _precompute_body,
        grid=(NB,),
        in_specs=[
            pl.BlockSpec((B, H), lambda i: (i, 0)),
            pl.BlockSpec((1, 1, B), lambda i: (i, 0, 0)),
            const((H, H)),
            const((1, H)),
        ],
        out_specs=[
            pl.BlockSpec((B, H), lambda i: (i, 0)),
            const((G, H)),
            const((G, 1)),
        ],
        out_shape=[
            jax.ShapeDtypeStruct((N, H), f32),
            jax.ShapeDtypeStruct((G, H), f32),
            jax.ShapeDtypeStruct((G, 1), f32),
        ],
    )(x, brow, gate_W1, b1r)

    repr_, rw1 = pl.pallas_call(
        _meanw1_body,
        grid=(1,),
        in_specs=[const((G, H)), const((G, 1)), const((H, H))],
        out_specs=[const((G, H)), const((G, H))],
        out_shape=[jax.ShapeDtypeStruct((G, H), f32),
                   jax.ShapeDtypeStruct((G, H), f32)],
    )(sums0, cnt, gate_W1)

    mesh = plsc.VectorSubcoreMesh(core_axis_name="c", subcore_axis_name="s")
    gatepass = pl.kernel(
        _sc_gatepass_body,
        out_type=jax.ShapeDtypeStruct((NC * G, H), f32),
        mesh=mesh,
        scratch_types=[
            pltpu.VMEM((CH,), jnp.int32),
            pltpu.VMEM((CH, H), f32),
            pltpu.VMEM((CH, H), f32),
            pltpu.VMEM((CH,), jnp.int32),
            pltpu.VMEM((CH, H), f32),
            pltpu.VMEM((CH, H), f32),
            pltpu.VMEM((CH, H), f32),
            pltpu.VMEM((CH, H), f32),
            pltpu.VMEM((H,), f32),
            pltpu.VMEM((16,), f32),
            pltpu.VMEM_SHARED((G, H), f32),
            pltpu.SemaphoreType.DMA,
            pltpu.SemaphoreType.DMA,
            pltpu.SemaphoreType.DMA,
        ],
    )

    gru = pl.pallas_call(
        _gru_body,
        grid=(1,),
        in_specs=[const((NC, G, H)), const((G, 1)), const((G, H)),
                  const((H, 3 * H)), const((H, 3 * H)),
                  const((1, 3 * H)), const((1, 3 * H)), const((H, H))],
        out_specs=[const((G, H)), const((G, H))],
        out_shape=[jax.ShapeDtypeStruct((G, H), f32),
                   jax.ShapeDtypeStruct((G, H), f32)],
    )

    for _ in range(2):
        wsums = gatepass(xw1, x, batch, rw1, w2flat, b2v, zeros_gh)
        repr_, rw1 = gru(wsums.reshape(NC, G, H), cnt, repr_,
                         wihT, whhT, bihr, bhhr, gate_W1)

    return repr_


# speedup vs baseline: 4.0231x; 2.0221x over previous
import functools

import jax
import jax.numpy as jnp
from jax import lax
from jax.experimental import pallas as pl
from jax.experimental.pallas import tpu as pltpu
from jax.experimental.pallas import tpu_sc as plsc

N = 100000
H = 128
G = 1024
B = 4000
NB = N // B
GC = 128
NGC = G // GC

NC = 2
NS = 16
NW = NC * NS
CH = 80

KTC = 19
COFF = (KTC * B) // CH
NCHS = (N - KTC * B) // CH
JFULL = NCHS // NW
TAIL = NCHS % NW
assert JFULL % 2 == 1 and (N - KTC * B) % CH == 0

_dot = functools.partial(jnp.dot, preferred_element_type=jnp.float32)


def _hilo(a):
    hi = a.astype(jnp.bfloat16)
    lo = (a - hi.astype(jnp.float32)).astype(jnp.bfloat16)
    return hi, lo


def _mm3(a, b):
    ah, al = _hilo(a)
    bh, bl = _hilo(b)
    return _dot(ah, bh) + (_dot(al, bh) + _dot(ah, bl))


def _precompute_body(x_ref, brow_ref, w1_ref, b1_ref,
                     xw1_ref, sums_ref, cnt_ref):
    i = pl.program_id(0)

    @pl.when(i == 0)
    def _():
        sums_ref[...] = jnp.zeros_like(sums_ref)
        cnt_ref[...] = jnp.zeros_like(cnt_ref)

    xb = x_ref[...]
    xw1_ref[...] = _mm3(xb, w1_ref[...]) + b1_ref[...]

    brow = brow_ref[0]
    bmin = brow_ref[0, 0, 0]
    bmax = brow_ref[0, 0, B - 1]
    xh, xl = _hilo(xb)
    for c in range(NGC):
        base = c * GC

        @pl.when((bmax >= base) & (bmin < base + GC))
        def _(base=base):
            iog = lax.broadcasted_iota(jnp.int32, (GC, B), 0) + base
            ohg = (iog == brow)
            ohb = ohg.astype(jnp.bfloat16)
            sums_ref[base:base + GC, :] += _dot(ohb, xh) + _dot(ohb, xl)
            cnt_ref[base:base + GC, :] += jnp.sum(
                ohg.astype(jnp.float32), axis=1, keepdims=True)


def _tc_gatepass_body(xw1_ref, x_ref, brow_ref, bcol_ref, rw1_ref,
                      w2_ref, b2_ref, wsums_ref, gath_ref):
    i = pl.program_id(0)

    @pl.when(i == 0)
    def _():
        wsums_ref[...] = jnp.zeros_like(wsums_ref)

    gath_ref[...] = jnp.zeros_like(gath_ref)
    bcol = bcol_ref[...]
    brow = brow_ref[0]
    bmin = brow_ref[0, 0, 0]
    bmax = brow_ref[0, 0, B - 1]

    for c in range(NGC):
        base = c * GC

        @pl.when((bmax >= base) & (bmin < base + GC))
        def _(base=base):
            ion = lax.broadcasted_iota(jnp.int32, (B, GC), 1) + base
            ohn = (ion == bcol).astype(jnp.bfloat16)
            rh, rl = _hilo(rw1_ref[base:base + GC, :])
            gath_ref[...] += _dot(ohn, rh) + _dot(ohn, rl)

    h = jnp.maximum(xw1_ref[...] + gath_ref[...], 0.0)
    z = jnp.sum(h * w2_ref[...], axis=1, keepdims=True) + b2_ref[0, 0]
    gate = jax.nn.sigmoid(z)
    w = x_ref[...] * gate
    wh, wl = _hilo(w)
    for c in range(NGC):
        base = c * GC

        @pl.when((bmax >= base) & (bmin < base + GC))
        def _(base=base):
            iog = lax.broadcasted_iota(jnp.int32, (GC, B), 0) + base
            ohg = (iog == brow).astype(jnp.bfloat16)
            wsums_ref[base:base + GC, :] += _dot(ohg, wh) + _dot(ohg, wl)


def _meanw1_body(sums_ref, cnt_ref, w1_ref, repr_ref, rw1_ref):
    mean = sums_ref[...] / jnp.maximum(cnt_ref[...], 1.0)
    repr_ref[...] = mean
    rw1_ref[...] = _mm3(mean, w1_ref[...])


def _gru_body(wsums_tc_ref, wsums_ref, cnt_ref, prev_ref, wihT_ref, whhT_ref,
              bih_ref, bhh_ref, w1_ref, repr_ref, rw1_ref):
    wsums = wsums_tc_ref[...] + wsums_ref[0] + wsums_ref[1]
    mean = wsums / jnp.maximum(cnt_ref[...], 1.0)
    prev = prev_ref[...]
    gi = _mm3(mean, wihT_ref[...]) + bih_ref[...]
    gh = _mm3(prev, whhT_ref[...]) + bhh_ref[...]
    r = jax.nn.sigmoid(gi[:, :H] + gh[:, :H])
    z = jax.nn.sigmoid(gi[:, H:2 * H] + gh[:, H:2 * H])
    n = jnp.tanh(gi[:, 2 * H:] + r * gh[:, 2 * H:])
    new = jnp.maximum((1.0 - z) * n + z * prev, 0.0)
    repr_ref[...] = new
    rw1_ref[...] = _mm3(new, w1_ref[...])


def _sc_gatepass_body(xw1_hbm, x_hbm, batch_hbm, rw1_hbm, w2_hbm, b2_hbm,
                      zeros_hbm, out_hbm,
                      idx_a, xw_a, x_a, idx_b, xw_b, x_b,
                      r_v, w_v, w2_v, b2_v, accum, sem_a, sem_b, sem_g):
    c = lax.axis_index("c")
    s = lax.axis_index("s")
    w = s * NC + c

    pltpu.sync_copy(zeros_hbm.at[pl.ds(s * (G // NS), G // NS)],
                    accum.at[pl.ds(s * (G // NS), G // NS)])
    pltpu.sync_copy(w2_hbm, w2_v)
    pltpu.sync_copy(b2_hbm, b2_v)
    plsc.subcore_barrier()
    b2vec = b2_v[...]
    lanes = lax.broadcasted_iota(jnp.int32, (16,), 0)
    rots = [(lanes + sh) % 16 for sh in (8, 4, 2, 1)]

    def _in_copies(j, idxv, xwv, xv, sem):
        base = (COFF + j * NW + w) * CH
        return (pltpu.make_async_copy(batch_hbm.at[pl.ds(base, CH)], idxv, sem),
                pltpu.make_async_copy(xw1_hbm.at[pl.ds(base, CH)], xwv, sem),
                pltpu.make_async_copy(x_hbm.at[pl.ds(base, CH)], xv, sem))

    def _start_in(j, idxv, xwv, xv, sem):
        for cp in _in_copies(j, idxv, xwv, xv, sem):
            cp.start()

    def _wait_in(j, idxv, xwv, xv, sem):
        for cp in _in_copies(j, idxv, xwv, xv, sem):
            cp.wait()

    def _process(idxv, xwv, xv):
        pltpu.async_copy(rw1_hbm.at[idxv], r_v, sem_g).wait()

        @plsc.parallel_loop(0, CH, 1, unroll=4)
        def _row(i):
            acc = jnp.zeros((16,), jnp.float32)
            for l in range(8):
                a = xwv[i, pl.ds(l * 16, 16)] + r_v[i, pl.ds(l * 16, 16)]
                acc = acc + jnp.maximum(a, 0.0) * w2_v[pl.ds(l * 16, 16)]
            for rot in rots:
                acc = acc + lax.gather(
                    acc, rot[:, None],
                    lax.GatherDimensionNumbers(
                        offset_dims=(), collapsed_slice_dims=(0,),
                        start_index_map=(0,)),
                    slice_sizes=(1,),
                    mode=lax.GatherScatterMode.PROMISE_IN_BOUNDS)
            z = acc + b2vec
            g = 1.0 / (1.0 + jnp.exp(-z))
            for l in range(8):
                w_v[i, pl.ds(l * 16, 16)] = xv[i, pl.ds(l * 16, 16)] * g
        pltpu.sync_copy(w_v, accum.at[idxv], add=True)

    _start_in(0, idx_a, xw_a, x_a, sem_a)

    def pair_body(k, carry):
        j0 = 2 * k
        _wait_in(j0, idx_a, xw_a, x_a, sem_a)
        _start_in(j0 + 1, idx_b, xw_b, x_b, sem_b)
        _process(idx_a, xw_a, x_a)
        _wait_in(j0 + 1, idx_b, xw_b, x_b, sem_b)
        _start_in(j0 + 2, idx_a, xw_a, x_a, sem_a)
        _process(idx_b, xw_b, x_b)
        return carry

    lax.fori_loop(0, (JFULL - 1) // 2, pair_body, 0)
    _wait_in(JFULL - 1, idx_a, xw_a, x_a, sem_a)
    _process(idx_a, xw_a, x_a)

    @pl.when(JFULL * NW + w < NCHS)
    def _():
        _start_in(JFULL, idx_b, xw_b, x_b, sem_b)
        _wait_in(JFULL, idx_b, xw_b, x_b, sem_b)
        _process(idx_b, xw_b, x_b)

    plsc.subcore_barrier()

    off = c * G + s * (G // NS)
    pltpu.sync_copy(accum.at[pl.ds(s * (G // NS), G // NS)],
                    out_hbm.at[pl.ds(off, G // NS)])


def kernel(x, batch, gate_W1, gate_b1, gate_W2, gate_b2,
           W_ih, W_hh, b_ih, b_hh):
    batch = batch.astype(jnp.int32)
    brow = batch.reshape(NB, 1, B)
    bcol = batch.reshape(N, 1)
    b1r = gate_b1.reshape(1, H)
    w2r = gate_W2.reshape(1, H)
    b2p = jnp.broadcast_to(gate_b2.reshape(1, 1), (1, GC))
    w2flat = gate_W2.reshape(H)
    b2v = jnp.broadcast_to(gate_b2.reshape(1), (16,))
    wihT = W_ih.T
    whhT = W_hh.T
    bihr = b_ih.reshape(1, 3 * H)
    bhhr = b_hh.reshape(1, 3 * H)
    zeros_gh = jnp.zeros((G, H), jnp.float32)

    f32 = jnp.float32
    const = lambda shape: pl.BlockSpec(shape, lambda i: tuple(0 for _ in shape))

    xw1, sums0, cnt = pl.pallas_call(
        _precompute_body,
        grid=(NB,),
        in_specs=[
            pl.BlockSpec((B, H), lambda i: (i, 0)),
            pl.BlockSpec((1, 1, B), lambda i: (i, 0, 0)),
            const((H, H)),
            const((1, H)),
        ],
        out_specs=[
            pl.BlockSpec((B, H), lambda i: (i, 0)),
            const((G, H)),
            const((G, 1)),
        ],
        out_shape=[
            jax.ShapeDtypeStruct((N, H), f32),
            jax.ShapeDtypeStruct((G, H), f32),
            jax.ShapeDtypeStruct((G, 1), f32),
        ],
    )(x, brow, gate_W1, b1r)

    repr_, rw1 = pl.pallas_call(
        _meanw1_body,
        grid=(1,),
        in_specs=[const((G, H)), const((G, 1)), const((H, H))],
        out_specs=[const((G, H)), const((G, H))],
        out_shape=[jax.ShapeDtypeStruct((G, H), f32),
                   jax.ShapeDtypeStruct((G, H), f32)],
    )(sums0, cnt, gate_W1)

    mesh = plsc.VectorSubcoreMesh(core_axis_name="c", subcore_axis_name="s")
    gatepass = pl.kernel(
        _sc_gatepass_body,
        out_type=jax.ShapeDtypeStruct((NC * G, H), f32),
        mesh=mesh,
        scratch_types=[
            pltpu.VMEM((CH,), jnp.int32),
            pltpu.VMEM((CH, H), f32),
            pltpu.VMEM((CH, H), f32),
            pltpu.VMEM((CH,), jnp.int32),
            pltpu.VMEM((CH, H), f32),
            pltpu.VMEM((CH, H), f32),
            pltpu.VMEM((CH, H), f32),
            pltpu.VMEM((CH, H), f32),
            pltpu.VMEM((H,), f32),
            pltpu.VMEM((16,), f32),
            pltpu.VMEM_SHARED((G, H), f32),
            pltpu.SemaphoreType.DMA,
            pltpu.SemaphoreType.DMA,
            pltpu.SemaphoreType.DMA,
        ],
    )

    tc_gatepass = pl.pallas_call(
        _tc_gatepass_body,
        grid=(KTC,),
        in_specs=[
            pl.BlockSpec((B, H), lambda i: (i, 0)),
            pl.BlockSpec((B, H), lambda i: (i, 0)),
            pl.BlockSpec((1, 1, B), lambda i: (i, 0, 0)),
            pl.BlockSpec((B, 1), lambda i: (i, 0)),
            const((G, H)),
            const((1, H)),
            const((1, GC)),
        ],
        out_specs=[const((G, H))],
        out_shape=[jax.ShapeDtypeStruct((G, H), f32)],
        scratch_shapes=[pltpu.VMEM((B, H), f32)],
    )

    gru = pl.pallas_call(
        _gru_body,
        grid=(1,),
        in_specs=[const((G, H)), const((NC, G, H)), const((G, 1)),
                  const((G, H)),
                  const((H, 3 * H)), const((H, 3 * H)),
                  const((1, 3 * H)), const((1, 3 * H)), const((H, H))],
        out_specs=[const((G, H)), const((G, H))],
        out_shape=[jax.ShapeDtypeStruct((G, H), f32),
                   jax.ShapeDtypeStruct((G, H), f32)],
    )

    for _ in range(2):
        wsums_sc = gatepass(xw1, x, batch, rw1, w2flat, b2v, zeros_gh)
        (wsums_tc,) = tc_gatepass(xw1, x, brow, bcol, rw1, w2r, b2p)
        repr_, rw1 = gru(wsums_tc, wsums_sc.reshape(NC, G, H), cnt, repr_,
                         wihT, whhT, bihr, bhhr, gate_W1)

    return repr_


# speedup vs baseline: 4.2073x; 1.0458x over previous
import functools

import jax
import jax.numpy as jnp
from jax import lax
from jax.experimental import pallas as pl
from jax.experimental.pallas import tpu as pltpu
from jax.experimental.pallas import tpu_sc as plsc

N = 100000
H = 128
G = 1024
B = 4000
NB = N // B
GC = 128
NGC = G // GC

NC = 2
NS = 16
NW = NC * NS
CH = 80

KTC = 22
COFF = (KTC * B) // CH
NCHS = (N - KTC * B) // CH
JFULL = NCHS // NW
TAIL = NCHS % NW
assert JFULL >= 2 and (N - KTC * B) % CH == 0

_dot = functools.partial(jnp.dot, preferred_element_type=jnp.float32)


def _hilo(a):
    hi = a.astype(jnp.bfloat16)
    lo = (a - hi.astype(jnp.float32)).astype(jnp.bfloat16)
    return hi, lo


def _mm3(a, b):
    ah, al = _hilo(a)
    bh, bl = _hilo(b)
    return _dot(ah, bh) + (_dot(al, bh) + _dot(ah, bl))


def _precompute_body(x_ref, brow_ref, w1_ref, b1_ref,
                     xw1_ref, sums_ref, cnt_ref):
    i = pl.program_id(0)

    @pl.when(i == 0)
    def _():
        sums_ref[...] = jnp.zeros_like(sums_ref)
        cnt_ref[...] = jnp.zeros_like(cnt_ref)

    xb = x_ref[...]
    xw1_ref[...] = _mm3(xb, w1_ref[...]) + b1_ref[...]

    brow = brow_ref[0]
    bmin = brow_ref[0, 0, 0]
    bmax = brow_ref[0, 0, B - 1]
    xh, xl = _hilo(xb)
    for c in range(NGC):
        base = c * GC

        @pl.when((bmax >= base) & (bmin < base + GC))
        def _(base=base):
            iog = lax.broadcasted_iota(jnp.int32, (GC, B), 0) + base
            ohg = (iog == brow)
            ohb = ohg.astype(jnp.bfloat16)
            sums_ref[base:base + GC, :] += _dot(ohb, xh) + _dot(ohb, xl)
            cnt_ref[base:base + GC, :] += jnp.sum(
                ohg.astype(jnp.float32), axis=1, keepdims=True)


def _tc_gatepass_body(xw1_ref, x_ref, brow_ref, bcol_ref, rw1_ref,
                      w2_ref, b2_ref, wsums_ref, gath_ref):
    i = pl.program_id(0)

    @pl.when(i == 0)
    def _():
        wsums_ref[...] = jnp.zeros_like(wsums_ref)

    gath_ref[...] = jnp.zeros_like(gath_ref)
    bcol = bcol_ref[...]
    brow = brow_ref[0]
    bmin = brow_ref[0, 0, 0]
    bmax = brow_ref[0, 0, B - 1]

    for c in range(NGC):
        base = c * GC

        @pl.when((bmax >= base) & (bmin < base + GC))
        def _(base=base):
            ion = lax.broadcasted_iota(jnp.int32, (B, GC), 1) + base
            ohn = (ion == bcol).astype(jnp.bfloat16)
            rh, rl = _hilo(rw1_ref[base:base + GC, :])
            gath_ref[...] += _dot(ohn, rh) + _dot(ohn, rl)

    h = jnp.maximum(xw1_ref[...] + gath_ref[...], 0.0)
    z = jnp.sum(h * w2_ref[...], axis=1, keepdims=True) + b2_ref[0, 0]
    gate = jax.nn.sigmoid(z)
    w = x_ref[...] * gate
    wh, wl = _hilo(w)
    for c in range(NGC):
        base = c * GC

        @pl.when((bmax >= base) & (bmin < base + GC))
        def _(base=base):
            iog = lax.broadcasted_iota(jnp.int32, (GC, B), 0) + base
            ohg = (iog == brow).astype(jnp.bfloat16)
            wsums_ref[base:base + GC, :] += _dot(ohg, wh) + _dot(ohg, wl)


def _meanw1_body(sums_ref, cnt_ref, w1_ref, repr_ref, rw1_ref):
    mean = sums_ref[...] / jnp.maximum(cnt_ref[...], 1.0)
    repr_ref[...] = mean
    rw1_ref[...] = _mm3(mean, w1_ref[...])


def _gru_body(wsums_tc_ref, wsums_ref, cnt_ref, prev_ref, wihT_ref, whhT_ref,
              bih_ref, bhh_ref, w1_ref, repr_ref, rw1_ref):
    wsums = wsums_tc_ref[...] + wsums_ref[0] + wsums_ref[1]
    mean = wsums / jnp.maximum(cnt_ref[...], 1.0)
    prev = prev_ref[...]
    gi = _mm3(mean, wihT_ref[...]) + bih_ref[...]
    gh = _mm3(prev, whhT_ref[...]) + bhh_ref[...]
    r = jax.nn.sigmoid(gi[:, :H] + gh[:, :H])
    z = jax.nn.sigmoid(gi[:, H:2 * H] + gh[:, H:2 * H])
    n = jnp.tanh(gi[:, 2 * H:] + r * gh[:, 2 * H:])
    new = jnp.maximum((1.0 - z) * n + z * prev, 0.0)
    repr_ref[...] = new
    rw1_ref[...] = _mm3(new, w1_ref[...])


def _sc_gatepass_body(xw1_hbm, x_hbm, batch_hbm, rw1_hbm, w2_hbm, b2_hbm,
                      zeros_hbm, out_hbm,
                      idx_a, xw_a, x_a, idx_b, xw_b, x_b,
                      r_v, w_v, w2_v, b2_v, accum, sem_a, sem_b, sem_g):
    c = lax.axis_index("c")
    s = lax.axis_index("s")
    w = s * NC + c

    pltpu.sync_copy(zeros_hbm.at[pl.ds(s * (G // NS), G // NS)],
                    accum.at[pl.ds(s * (G // NS), G // NS)])
    pltpu.sync_copy(w2_hbm, w2_v)
    pltpu.sync_copy(b2_hbm, b2_v)
    plsc.subcore_barrier()
    b2vec = b2_v[...]
    lanes = lax.broadcasted_iota(jnp.int32, (16,), 0)
    rots = [(lanes + sh) % 16 for sh in (8, 4, 2, 1)]

    def _in_copies(j, idxv, xwv, xv, sem):
        base = (COFF + j * NW + w) * CH
        return (pltpu.make_async_copy(batch_hbm.at[pl.ds(base, CH)], idxv, sem),
                pltpu.make_async_copy(xw1_hbm.at[pl.ds(base, CH)], xwv, sem),
                pltpu.make_async_copy(x_hbm.at[pl.ds(base, CH)], xv, sem))

    def _start_in(j, idxv, xwv, xv, sem):
        for cp in _in_copies(j, idxv, xwv, xv, sem):
            cp.start()

    def _wait_in(j, idxv, xwv, xv, sem):
        for cp in _in_copies(j, idxv, xwv, xv, sem):
            cp.wait()

    def _process(idxv, xwv, xv):
        pltpu.async_copy(rw1_hbm.at[idxv], r_v, sem_g).wait()

        @plsc.parallel_loop(0, CH, 1, unroll=8)
        def _row(i):
            acc = jnp.zeros((16,), jnp.float32)
            for l in range(8):
                a = xwv[i, pl.ds(l * 16, 16)] + r_v[i, pl.ds(l * 16, 16)]
                acc = acc + jnp.maximum(a, 0.0) * w2_v[pl.ds(l * 16, 16)]
            for rot in rots:
                acc = acc + lax.gather(
                    acc, rot[:, None],
                    lax.GatherDimensionNumbers(
                        offset_dims=(), collapsed_slice_dims=(0,),
                        start_index_map=(0,)),
                    slice_sizes=(1,),
                    mode=lax.GatherScatterMode.PROMISE_IN_BOUNDS)
            z = acc + b2vec
            g = 1.0 / (1.0 + jnp.exp(-z))
            for l in range(8):
                w_v[i, pl.ds(l * 16, 16)] = xv[i, pl.ds(l * 16, 16)] * g
        pltpu.sync_copy(w_v, accum.at[idxv], add=True)

    _start_in(0, idx_a, xw_a, x_a, sem_a)

    def pair_body(k, carry):
        j0 = 2 * k
        _wait_in(j0, idx_a, xw_a, x_a, sem_a)
        _start_in(j0 + 1, idx_b, xw_b, x_b, sem_b)
        _process(idx_a, xw_a, x_a)
        _wait_in(j0 + 1, idx_b, xw_b, x_b, sem_b)
        _start_in(j0 + 2, idx_a, xw_a, x_a, sem_a)
        _process(idx_b, xw_b, x_b)
        return carry

    if JFULL % 2 == 1:
        lax.fori_loop(0, (JFULL - 1) // 2, pair_body, 0)
        _wait_in(JFULL - 1, idx_a, xw_a, x_a, sem_a)
        _process(idx_a, xw_a, x_a)
        tail_bufs = (idx_b, xw_b, x_b, sem_b)
    else:
        lax.fori_loop(0, (JFULL - 2) // 2, pair_body, 0)
        _wait_in(JFULL - 2, idx_a, xw_a, x_a, sem_a)
        _start_in(JFULL - 1, idx_b, xw_b, x_b, sem_b)
        _process(idx_a, xw_a, x_a)
        _wait_in(JFULL - 1, idx_b, xw_b, x_b, sem_b)
        _process(idx_b, xw_b, x_b)
        tail_bufs = (idx_a, xw_a, x_a, sem_a)

    @pl.when(JFULL * NW + w < NCHS)
    def _():
        _start_in(JFULL, *tail_bufs)
        _wait_in(JFULL, *tail_bufs)
        _process(tail_bufs[0], tail_bufs[1], tail_bufs[2])

    plsc.subcore_barrier()

    off = c * G + s * (G // NS)
    pltpu.sync_copy(accum.at[pl.ds(s * (G // NS), G // NS)],
                    out_hbm.at[pl.ds(off, G // NS)])


def kernel(x, batch, gate_W1, gate_b1, gate_W2, gate_b2,
           W_ih, W_hh, b_ih, b_hh):
    batch = batch.astype(jnp.int32)
    brow = batch.reshape(NB, 1, B)
    bcol = batch.reshape(N, 1)
    b1r = gate_b1.reshape(1, H)
    w2r = gate_W2.reshape(1, H)
    b2p = jnp.broadcast_to(gate_b2.reshape(1, 1), (1, GC))
    w2flat = gate_W2.reshape(H)
    b2v = jnp.broadcast_to(gate_b2.reshape(1), (16,))
    wihT = W_ih.T
    whhT = W_hh.T
    bihr = b_ih.reshape(1, 3 * H)
    bhhr = b_hh.reshape(1, 3 * H)
    zeros_gh = jnp.zeros((G, H), jnp.float32)

    f32 = jnp.float32
    const = lambda shape: pl.BlockSpec(shape, lambda i: tuple(0 for _ in shape))

    xw1, sums0, cnt = pl.pallas_call(
        _precompute_body,
        grid=(NB,),
        in_specs=[
            pl.BlockSpec((B, H), lambda i: (i, 0)),
            pl.BlockSpec((1, 1, B), lambda i: (i, 0, 0)),
            const((H, H)),
            const((1, H)),
        ],
        out_specs=[
            pl.BlockSpec((B, H), lambda i: (i, 0)),
            const((G, H)),
            const((G, 1)),
        ],
        out_shape=[
            jax.ShapeDtypeStruct((N, H), f32),
            jax.ShapeDtypeStruct((G, H), f32),
            jax.ShapeDtypeStruct((G, 1), f32),
        ],
    )(x, brow, gate_W1, b1r)

    repr_, rw1 = pl.pallas_call(
        _meanw1_body,
        grid=(1,),
        in_specs=[const((G, H)), const((G, 1)), const((H, H))],
        out_specs=[const((G, H)), const((G, H))],
        out_shape=[jax.ShapeDtypeStruct((G, H), f32),
                   jax.ShapeDtypeStruct((G, H), f32)],
    )(sums0, cnt, gate_W1)

    mesh = plsc.VectorSubcoreMesh(core_axis_name="c", subcore_axis_name="s")
    gatepass = pl.kernel(
        _sc_gatepass_body,
        out_type=jax.ShapeDtypeStruct((NC * G, H), f32),
        mesh=mesh,
        scratch_types=[
            pltpu.VMEM((CH,), jnp.int32),
            pltpu.VMEM((CH, H), f32),
            pltpu.VMEM((CH, H), f32),
            pltpu.VMEM((CH,), jnp.int32),
            pltpu.VMEM((CH, H), f32),
            pltpu.VMEM((CH, H), f32),
            pltpu.VMEM((CH, H), f32),
            pltpu.VMEM((CH, H), f32),
            pltpu.VMEM((H,), f32),
            pltpu.VMEM((16,), f32),
            pltpu.VMEM_SHARED((G, H), f32),
            pltpu.SemaphoreType.DMA,
            pltpu.SemaphoreType.DMA,
            pltpu.SemaphoreType.DMA,
        ],
    )

    tc_gatepass = pl.pallas_call(
        _tc_gatepass_body,
        grid=(KTC,),
        in_specs=[
            pl.BlockSpec((B, H), lambda i: (i, 0)),
            pl.BlockSpec((B, H), lambda i: (i, 0)),
            pl.BlockSpec((1, 1, B), lambda i: (i, 0, 0)),
            pl.BlockSpec((B, 1), lambda i: (i, 0)),
            const((G, H)),
            const((1, H)),
            const((1, GC)),
        ],
        out_specs=[const((G, H))],
        out_shape=[jax.ShapeDtypeStruct((G, H), f32)],
        scratch_shapes=[pltpu.VMEM((B, H), f32)],
    )

    gru = pl.pallas_call(
        _gru_body,
        grid=(1,),
        in_specs=[const((G, H)), const((NC, G, H)), const((G, 1)),
                  const((G, H)),
                  const((H, 3 * H)), const((H, 3 * H)),
                  const((1, 3 * H)), const((1, 3 * H)), const((H, H))],
        out_specs=[const((G, H)), const((G, H))],
        out_shape=[jax.ShapeDtypeStruct((G, H), f32),
                   jax.ShapeDtypeStruct((G, H), f32)],
    )

    for _ in range(2):
        wsums_sc = gatepass(xw1, x, batch, rw1, w2flat, b2v, zeros_gh)
        (wsums_tc,) = tc_gatepass(xw1, x, brow, bcol, rw1, w2r, b2p)
        repr_, rw1 = gru(wsums_tc, wsums_sc.reshape(NC, G, H), cnt, repr_,
                         wihT, whhT, bihr, bhhr, gate_W1)

    return repr_


# speedup vs baseline: 4.3805x; 1.0412x over previous
import functools

import jax
import jax.numpy as jnp
from jax import lax
from jax.experimental import pallas as pl
from jax.experimental.pallas import tpu as pltpu
from jax.experimental.pallas import tpu_sc as plsc

N = 100000
H = 128
G = 1024
B = 4000
NB = N // B
GC = 128
NGC = G // GC

NC = 2
NS = 16
NW = NC * NS
CH = 80

KTC = 22
COFF = (KTC * B) // CH
NCHS = (N - KTC * B) // CH
JFULL = NCHS // NW
TAIL = NCHS % NW
assert JFULL >= 2 and (N - KTC * B) % CH == 0

_dot = functools.partial(jnp.dot, preferred_element_type=jnp.float32)


def _hilo(a):
    hi = a.astype(jnp.bfloat16)
    lo = (a - hi.astype(jnp.float32)).astype(jnp.bfloat16)
    return hi, lo


def _mm3(a, b):
    ah, al = _hilo(a)
    bh, bl = _hilo(b)
    return _dot(ah, bh) + (_dot(al, bh) + _dot(ah, bl))


def _precompute_body(x_ref, brow_ref, w1_ref, b1_ref,
                     xw1_ref, sums_ref, cnt_ref):
    i = pl.program_id(0)

    @pl.when(i == 0)
    def _():
        sums_ref[...] = jnp.zeros_like(sums_ref)
        cnt_ref[...] = jnp.zeros_like(cnt_ref)

    xb = x_ref[...]
    xw1_ref[...] = _mm3(xb, w1_ref[...]) + b1_ref[...]

    brow = brow_ref[0]
    bmin = brow_ref[0, 0, 0]
    bmax = brow_ref[0, 0, B - 1]
    xh, xl = _hilo(xb)
    for c in range(NGC):
        base = c * GC

        @pl.when((bmax >= base) & (bmin < base + GC))
        def _(base=base):
            iog = lax.broadcasted_iota(jnp.int32, (GC, B), 0) + base
            ohg = (iog == brow)
            ohb = ohg.astype(jnp.bfloat16)
            sums_ref[base:base + GC, :] += _dot(ohb, xh) + _dot(ohb, xl)
            cnt_ref[base:base + GC, :] += jnp.sum(
                ohg.astype(jnp.float32), axis=1, keepdims=True)


def _tc_gatepass_body(xw1_ref, x_ref, brow_ref, bcol_ref, rw1_ref,
                      w2_ref, b2_ref, wsums_ref, gath_ref):
    i = pl.program_id(0)

    @pl.when(i == 0)
    def _():
        wsums_ref[...] = jnp.zeros_like(wsums_ref)

    gath_ref[...] = jnp.zeros_like(gath_ref)
    bcol = bcol_ref[...]
    brow = brow_ref[0]
    bmin = brow_ref[0, 0, 0]
    bmax = brow_ref[0, 0, B - 1]

    for c in range(NGC):
        base = c * GC

        @pl.when((bmax >= base) & (bmin < base + GC))
        def _(base=base):
            ion = lax.broadcasted_iota(jnp.int32, (B, GC), 1) + base
            ohn = (ion == bcol).astype(jnp.bfloat16)
            rh, rl = _hilo(rw1_ref[base:base + GC, :])
            gath_ref[...] += _dot(ohn, rh) + _dot(ohn, rl)

    h = jnp.maximum(xw1_ref[...] + gath_ref[...], 0.0)
    z = jnp.sum(h * w2_ref[...], axis=1, keepdims=True) + b2_ref[0, 0]
    gate = jax.nn.sigmoid(z)
    w = x_ref[...] * gate
    wh, wl = _hilo(w)
    for c in range(NGC):
        base = c * GC

        @pl.when((bmax >= base) & (bmin < base + GC))
        def _(base=base):
            iog = lax.broadcasted_iota(jnp.int32, (GC, B), 0) + base
            ohg = (iog == brow).astype(jnp.bfloat16)
            wsums_ref[base:base + GC, :] += _dot(ohg, wh) + _dot(ohg, wl)


def _meanw1_body(sums_ref, cnt_ref, w1_ref, repr_ref, rw1_ref):
    mean = sums_ref[...] / jnp.maximum(cnt_ref[...], 1.0)
    repr_ref[...] = mean
    rw1_ref[...] = _mm3(mean, w1_ref[...])


def _gru_body(wsums_tc_ref, wsums_ref, cnt_ref, prev_ref, wihT_ref, whhT_ref,
              bih_ref, bhh_ref, w1_ref, repr_ref, rw1_ref):
    wsums = wsums_tc_ref[...] + wsums_ref[0] + wsums_ref[1]
    mean = wsums / jnp.maximum(cnt_ref[...], 1.0)
    prev = prev_ref[...]
    gi = _mm3(mean, wihT_ref[...]) + bih_ref[...]
    gh = _mm3(prev, whhT_ref[...]) + bhh_ref[...]
    r = jax.nn.sigmoid(gi[:, :H] + gh[:, :H])
    z = jax.nn.sigmoid(gi[:, H:2 * H] + gh[:, H:2 * H])
    n = jnp.tanh(gi[:, 2 * H:] + r * gh[:, 2 * H:])
    new = jnp.maximum((1.0 - z) * n + z * prev, 0.0)
    repr_ref[...] = new
    rw1_ref[...] = _mm3(new, w1_ref[...])


def _sc_gatepass_body(xw1_hbm, x_hbm, batch_hbm, rw1_hbm, w2_hbm, b2_hbm,
                      zeros_hbm, out_hbm,
                      idx_a, xw_a, x_a, idx_b, xw_b, x_b, r_a, r_b,
                      w_v, w2_v, b2_v, accum,
                      sem_ia, sem_xa, sem_ga, sem_ib, sem_xb, sem_gb):
    c = lax.axis_index("c")
    s = lax.axis_index("s")
    w = s * NC + c

    pltpu.sync_copy(zeros_hbm.at[pl.ds(s * (G // NS), G // NS)],
                    accum.at[pl.ds(s * (G // NS), G // NS)])
    pltpu.sync_copy(w2_hbm, w2_v)
    pltpu.sync_copy(b2_hbm, b2_v)
    plsc.subcore_barrier()
    b2vec = b2_v[...]
    lanes = lax.broadcasted_iota(jnp.int32, (16,), 0)
    rots = [(lanes + sh) % 16 for sh in (8, 4, 2, 1)]

    sets = ((idx_a, xw_a, x_a, r_a, sem_ia, sem_xa, sem_ga),
            (idx_b, xw_b, x_b, r_b, sem_ib, sem_xb, sem_gb))

    def _base(j):
        return (COFF + j * NW + w) * CH

    def _idx_copy(j, S):
        return pltpu.make_async_copy(batch_hbm.at[pl.ds(_base(j), CH)],
                                     S[0], S[4])

    def _xx_copies(j, S):
        return (pltpu.make_async_copy(xw1_hbm.at[pl.ds(_base(j), CH)],
                                      S[1], S[5]),
                pltpu.make_async_copy(x_hbm.at[pl.ds(_base(j), CH)],
                                      S[2], S[5]))

    def _start_in(j, S):
        _idx_copy(j, S).start()
        for cp in _xx_copies(j, S):
            cp.start()

    def _start_gather(j, S):
        _idx_copy(j, S).wait()
        pltpu.make_async_copy(rw1_hbm.at[S[0]], S[3], S[6]).start()

    def _process(j, S):
        for cp in _xx_copies(j, S):
            cp.wait()
        pltpu.make_async_copy(rw1_hbm.at[S[0]], S[3], S[6]).wait()
        idxv, xwv, xv, rv = S[0], S[1], S[2], S[3]

        @plsc.parallel_loop(0, CH, 1, unroll=8)
        def _row(i):
            acc = jnp.zeros((16,), jnp.float32)
            for l in range(8):
                a = xwv[i, pl.ds(l * 16, 16)] + rv[i, pl.ds(l * 16, 16)]
                acc = acc + jnp.maximum(a, 0.0) * w2_v[pl.ds(l * 16, 16)]
            for rot in rots:
                acc = acc + lax.gather(
                    acc, rot[:, None],
                    lax.GatherDimensionNumbers(
                        offset_dims=(), collapsed_slice_dims=(0,),
                        start_index_map=(0,)),
                    slice_sizes=(1,),
                    mode=lax.GatherScatterMode.PROMISE_IN_BOUNDS)
            z = acc + b2vec
            g = 1.0 / (1.0 + jnp.exp(-z))
            for l in range(8):
                w_v[i, pl.ds(l * 16, 16)] = xv[i, pl.ds(l * 16, 16)] * g
        pltpu.sync_copy(w_v, accum.at[idxv], add=True)

    _start_in(0, sets[0])
    _start_gather(0, sets[0])
    if JFULL > 1:
        _start_in(1, sets[1])
    for j in range(JFULL):
        S, S2 = sets[j % 2], sets[(j + 1) % 2]
        if j + 1 < JFULL:
            _start_gather(j + 1, S2)
        elif TAIL > 0:

            @pl.when(w < TAIL)
            def _(S2=S2):
                _start_gather(JFULL, S2)

        _process(j, S)
        if j + 2 < JFULL:
            _start_in(j + 2, S)
        elif j + 2 == JFULL and TAIL > 0:

            @pl.when(w < TAIL)
            def _(S=S):
                _start_in(JFULL, S)

    if TAIL > 0:

        @pl.when(w < TAIL)
        def _():
            _process(JFULL, sets[JFULL % 2])

    plsc.subcore_barrier()

    off = c * G + s * (G // NS)
    pltpu.sync_copy(accum.at[pl.ds(s * (G // NS), G // NS)],
                    out_hbm.at[pl.ds(off, G // NS)])


def kernel(x, batch, gate_W1, gate_b1, gate_W2, gate_b2,
           W_ih, W_hh, b_ih, b_hh):
    batch = batch.astype(jnp.int32)
    brow = batch.reshape(NB, 1, B)
    bcol = batch.reshape(N, 1)
    b1r = gate_b1.reshape(1, H)
    w2r = gate_W2.reshape(1, H)
    b2p = jnp.broadcast_to(gate_b2.reshape(1, 1), (1, GC))
    w2flat = gate_W2.reshape(H)
    b2v = jnp.broadcast_to(gate_b2.reshape(1), (16,))
    wihT = W_ih.T
    whhT = W_hh.T
    bihr = b_ih.reshape(1, 3 * H)
    bhhr = b_hh.reshape(1, 3 * H)
    zeros_gh = jnp.zeros((G, H), jnp.float32)

    f32 = jnp.float32
    const = lambda shape: pl.BlockSpec(shape, lambda i: tuple(0 for _ in shape))

    xw1, sums0, cnt = pl.pallas_call(
        _precompute_body,
        grid=(NB,),
        in_specs=[
            pl.BlockSpec((B, H), lambda i: (i, 0)),
            pl.BlockSpec((1, 1, B), lambda i: (i, 0, 0)),
            const((H, H)),
            const((1, H)),
        ],
        out_specs=[
            pl.BlockSpec((B, H), lambda i: (i, 0)),
            const((G, H)),
            const((G, 1)),
        ],
        out_shape=[
            jax.ShapeDtypeStruct((N, H), f32),
            jax.ShapeDtypeStruct((G, H), f32),
            jax.ShapeDtypeStruct((G, 1), f32),
        ],
    )(x, brow, gate_W1, b1r)

    repr_, rw1 = pl.pallas_call(
        _meanw1_body,
        grid=(1,),
        in_specs=[const((G, H)), const((G, 1)), const((H, H))],
        out_specs=[const((G, H)), const((G, H))],
        out_shape=[jax.ShapeDtypeStruct((G, H), f32),
                   jax.ShapeDtypeStruct((G, H), f32)],
    )(sums0, cnt, gate_W1)

    mesh = plsc.VectorSubcoreMesh(core_axis_name="c", subcore_axis_name="s")
    gatepass = pl.kernel(
        _sc_gatepass_body,
        out_type=jax.ShapeDtypeStruct((NC * G, H), f32),
        mesh=mesh,
        scratch_types=[
            pltpu.VMEM((CH,), jnp.int32),
            pltpu.VMEM((CH, H), f32),
            pltpu.VMEM((CH, H), f32),
            pltpu.VMEM((CH,), jnp.int32),
            pltpu.VMEM((CH, H), f32),
            pltpu.VMEM((CH, H), f32),
            pltpu.VMEM((CH, H), f32),
            pltpu.VMEM((CH, H), f32),
            pltpu.VMEM((CH, H), f32),
            pltpu.VMEM((H,), f32),
            pltpu.VMEM((16,), f32),
            pltpu.VMEM_SHARED((G, H), f32),
            pltpu.SemaphoreType.DMA,
            pltpu.SemaphoreType.DMA,
            pltpu.SemaphoreType.DMA,
            pltpu.SemaphoreType.DMA,
            pltpu.SemaphoreType.DMA,
            pltpu.SemaphoreType.DMA,
        ],
    )

    tc_gatepass = pl.pallas_call(
        _tc_gatepass_body,
        grid=(KTC,),
        in_specs=[
            pl.BlockSpec((B, H), lambda i: (i, 0)),
            pl.BlockSpec((B, H), lambda i: (i, 0)),
            pl.BlockSpec((1, 1, B), lambda i: (i, 0, 0)),
            pl.BlockSpec((B, 1), lambda i: (i, 0)),
            const((G, H)),
            const((1, H)),
            const((1, GC)),
        ],
        out_specs=[const((G, H))],
        out_shape=[jax.ShapeDtypeStruct((G, H), f32)],
        scratch_shapes=[pltpu.VMEM((B, H), f32)],
    )

    gru = pl.pallas_call(
        _gru_body,
        grid=(1,),
        in_specs=[const((G, H)), const((NC, G, H)), const((G, 1)),
                  const((G, H)),
                  const((H, 3 * H)), const((H, 3 * H)),
                  const((1, 3 * H)), const((1, 3 * H)), const((H, H))],
        out_specs=[const((G, H)), const((G, H))],
        out_shape=[jax.ShapeDtypeStruct((G, H), f32),
                   jax.ShapeDtypeStruct((G, H), f32)],
    )

    for _ in range(2):
        wsums_sc = gatepass(xw1, x, batch, rw1, w2flat, b2v, zeros_gh)
        (wsums_tc,) = tc_gatepass(xw1, x, brow, bcol, rw1, w2r, b2p)
        repr_, rw1 = gru(wsums_tc, wsums_sc.reshape(NC, G, H), cnt, repr_,
                         wihT, whhT, bihr, bhhr, gate_W1)

    return repr_
